# Initial kernel scaffold; baseline (speedup 1.0000x reference)
import jax
import jax.numpy as jnp
from jax.experimental import pallas as pl


def kernel(theta, T_slices, x, a, prelu_alpha, W, b):
    raise NotImplementedError("write your pallas kernel here")


# speedup vs baseline: 2.6152x; 2.6152x over previous
import jax
import jax.numpy as jnp
from jax.experimental import pallas as pl
from jax.experimental.pallas import tpu as pltpu

K, N, D_IN, D_OUT = 8, 2048, 128, 128
BLK = 256


def _body(theta_ref, t_ref, a_ref, x_ref, wt_ref, alpha_ref, b_ref, o_ref):
    acc = theta_ref[0] * t_ref[0]
    for k in range(1, K):
        acc = acc + theta_ref[k] * t_ref[k]
    q = acc * a_ref[...]
    h = jnp.dot(q, x_ref[...], preferred_element_type=jnp.float32)
    h = jnp.where(h >= 0.0, h, alpha_ref[...] * h)
    o_ref[...] = jnp.dot(h, wt_ref[...], preferred_element_type=jnp.float32) + b_ref[...]


@jax.jit
def kernel(theta, T_slices, x, a, prelu_alpha, W, b):
    wt = W.T
    alpha = prelu_alpha.reshape(1, D_IN)
    bias = b.reshape(1, D_OUT)
    return pl.pallas_call(
        _body,
        grid=(N // BLK,),
        in_specs=[
            pl.BlockSpec(memory_space=pltpu.SMEM),
            pl.BlockSpec((K, BLK, N), lambda i: (0, i, 0)),
            pl.BlockSpec((BLK, N), lambda i: (i, 0)),
            pl.BlockSpec((N, D_IN), lambda i: (0, 0)),
            pl.BlockSpec((D_IN, D_OUT), lambda i: (0, 0)),
            pl.BlockSpec((1, D_IN), lambda i: (0, 0)),
            pl.BlockSpec((1, D_OUT), lambda i: (0, 0)),
        ],
        out_specs=pl.BlockSpec((BLK, D_OUT), lambda i: (i, 0)),
        out_shape=jax.ShapeDtypeStruct((N, D_OUT), jnp.float32),
    )(theta, T_slices, a, x, wt, alpha, bias)


# speedup vs baseline: 2.7906x; 1.0671x over previous
import jax
import jax.numpy as jnp
from jax.experimental import pallas as pl
from jax.experimental.pallas import tpu as pltpu

K, N, D_IN, D_OUT = 8, 2048, 128, 128
BLK = 128


def _body(theta_ref, t_ref, a_ref, x_ref, wt_ref, alpha_ref, b_ref, o_ref):
    acc = theta_ref[0] * t_ref[0]
    for k in range(1, K):
        acc = acc + theta_ref[k] * t_ref[k]
    q = acc * a_ref[...]
    h = jnp.dot(q, x_ref[...], preferred_element_type=jnp.float32)
    h = jnp.where(h >= 0.0, h, alpha_ref[...] * h)
    o_ref[...] = jnp.dot(h, wt_ref[...], preferred_element_type=jnp.float32) + b_ref[...]


@jax.jit
def kernel(theta, T_slices, x, a, prelu_alpha, W, b):
    wt = W.T
    alpha = prelu_alpha.reshape(1, D_IN)
    bias = b.reshape(1, D_OUT)
    return pl.pallas_call(
        _body,
        grid=(N // BLK,),
        in_specs=[
            pl.BlockSpec(memory_space=pltpu.SMEM),
            pl.BlockSpec((K, BLK, N), lambda i: (0, i, 0)),
            pl.BlockSpec((BLK, N), lambda i: (i, 0)),
            pl.BlockSpec((N, D_IN), lambda i: (0, 0)),
            pl.BlockSpec((D_IN, D_OUT), lambda i: (0, 0)),
            pl.BlockSpec((1, D_IN), lambda i: (0, 0)),
            pl.BlockSpec((1, D_OUT), lambda i: (0, 0)),
        ],
        out_specs=pl.BlockSpec((BLK, D_OUT), lambda i: (i, 0)),
        out_shape=jax.ShapeDtypeStruct((N, D_OUT), jnp.float32),
    )(theta, T_slices, a, x, wt, alpha, bias)
